# baseline (speedup 1.0000x reference)
import jax
import jax.numpy as jnp
from jax.experimental import pallas as pl

SEQ_BLOCK = 256


def _add_kernel(x_ref, pos_ref, o_ref):
    o_ref[...] = x_ref[...] + pos_ref[...]


def kernel(x, pos_table):
    B, S, D = x.shape
    num_s = S // SEQ_BLOCK
    return pl.pallas_call(
        _add_kernel,
        grid=(num_s,),
        in_specs=[
            pl.BlockSpec((B, SEQ_BLOCK, D), lambda s: (0, s, 0)),
            pl.BlockSpec((SEQ_BLOCK, D), lambda s: (s, 0)),
        ],
        out_specs=pl.BlockSpec((B, SEQ_BLOCK, D), lambda s: (0, s, 0)),
        out_shape=jax.ShapeDtypeStruct((B, S, D), x.dtype),
    )(x, pos_table)
